# baseline (speedup 1.0000x reference)
import functools

import jax
import jax.numpy as jnp
from jax import lax
from jax.experimental import pallas as pl
from jax.experimental.pallas import tpu as pltpu, tpu_sc as plsc

N = 10000
E = 320000
C = 128

NC, NS, L = 2, 16, 16
NW = NC * NS
B = 64
K0 = 152
K1 = 176
K = (K0 + K1) // 2
EPAD = NS * B * (K0 + K1)
PAD_N = 10112
ROWS_PER_TILE = PAD_N // NS
_ZCHUNKS = (128, 128, 128, 128, 120)


def _project_body(x_ref, wl_ref, wr_ref, xl_ref, xr_ref):
    xb = x_ref[...]
    xl_ref[...] = jnp.dot(xb, wl_ref[...], preferred_element_type=jnp.float32)
    xr_ref[...] = jnp.dot(xb, wr_ref[...], preferred_element_type=jnp.float32)


def _project(x, W_l, W_r):
    blk = 1000
    return pl.pallas_call(
        _project_body,
        grid=(N // blk,),
        in_specs=[
            pl.BlockSpec((blk, C), lambda i: (i, 0)),
            pl.BlockSpec((C, C), lambda i: (0, 0)),
            pl.BlockSpec((C, C), lambda i: (0, 0)),
        ],
        out_specs=[
            pl.BlockSpec((blk, C), lambda i: (i, 0)),
            pl.BlockSpec((blk, C), lambda i: (i, 0)),
        ],
        out_shape=[
            jax.ShapeDtypeStruct((N, C), jnp.float32),
            jax.ShapeDtypeStruct((N, C), jnp.float32),
        ],
    )(x, W_l, W_r)


def _sc_body(xl_hbm, xr_hbm, src_hbm, dst_hbm, att_hbm,
             acc_out, den_out,
             acc_sh, den_sh,
             xlb0, xlb1, xrb0, xrb1,
             sidx0, sidx1, didxg0, didxg1, didxs0, didxs1,
             drb0, drb1, attv,
             sem_g0, sem_g1, sem_i0, sem_i1, sem_s0, sem_s1):
    cid = lax.axis_index("c")
    sid = lax.axis_index("s")
    wid = cid * NS + sid

    zv = jnp.zeros((L,), jnp.float32)
    lane = lax.iota(jnp.int32, L)

    XLB, XRB = (xlb0, xlb1), (xrb0, xrb1)
    SIDX, DIDXG, DIDXS = (sidx0, sidx1), (didxg0, didxg1), (didxs0, didxs1)
    DRB = (drb0, drb1)
    SEMG, SEMI, SEMS = (sem_g0, sem_g1), (sem_i0, sem_i1), (sem_s0, sem_s1)

    def _zero(r, _):
        for c in range(C // L):
            xlb0[r, pl.ds(c * L, L)] = zv
        drb0[r, :] = zv
        return 0
    lax.fori_loop(0, B, _zero, 0)
    base_row = sid * ROWS_PER_TILE
    for t in range(10):
        sz = 64 if t < 9 else 56
        pltpu.sync_copy(xlb0.at[pl.ds(0, sz)],
                        acc_sh.at[pl.ds(base_row + t * 64, sz)])
        pltpu.sync_copy(drb0.at[pl.ds(0, sz)],
                        den_sh.at[pl.ds(base_row + t * 64, sz)])
    pltpu.sync_copy(att_hbm, attv)
    plsc.subcore_barrier()

    att_regs = [attv[pl.ds(c * L, L)] for c in range(C // L)]
    ebase = jnp.where(cid == 0, sid * (K0 * B),
                      NS * (K0 * B) + sid * (K1 * B))
    khalf = jnp.where(cid == 0, K0 // 2, K1 // 2)

    def _issue_gather(b, off):
        pltpu.async_copy(xl_hbm.at[SIDX[b]], XLB[b], SEMG[b])
        pltpu.async_copy(xr_hbm.at[DIDXG[b]], XRB[b], SEMG[b])
        del off

    def _wait_gather(b):
        pltpu.make_async_copy(xl_hbm.at[SIDX[b]], XLB[b], SEMG[b]).wait()
        pltpu.make_async_copy(xr_hbm.at[DIDXG[b]], XRB[b], SEMG[b]).wait()

    for b in (0, 1):
        pltpu.sync_copy(src_hbm.at[pl.ds(ebase + b * B, B)], SIDX[b])
        pltpu.sync_copy(dst_hbm.at[pl.ds(ebase + b * B, B)], DIDXG[b])
        _issue_gather(b, None)

    def _compute(b):
        xlb, xrb, drb = XLB[b], XRB[b], DRB[b]

        def _edge(e, _):
            xl_r = [xlb[e, pl.ds(c * L, L)] for c in range(C // L)]
            acc = zv
            for c in range(C // L):
                v = xl_r[c] + xrb[e, pl.ds(c * L, L)]
                v = jnp.maximum(v, 0.2 * v)
                acc = acc + v * att_regs[c]
            for h in (8, 4, 2, 1):
                acc = acc + acc.at[lane ^ h].get(mode="promise_in_bounds")
            wv = jnp.exp(acc)
            for c in range(C // L):
                xlb[e, pl.ds(c * L, L)] = xl_r[c] * wv
            drb[e, :] = jnp.where(lane == 0, wv, 0.0)
            return 0
        lax.fori_loop(0, B, _edge, 0, unroll=4)

    def _kk(kk, _):
        for b in (0, 1):
            k = 2 * kk + b
            _wait_gather(b)
            for j in range(B // L):
                DIDXS[b][pl.ds(j * L, L)] = DIDXG[b][pl.ds(j * L, L)]
            off2 = ebase + (k + 2) * B
            @pl.when(kk < khalf - 1)
            def _():
                pltpu.async_copy(src_hbm.at[pl.ds(off2, B)], SIDX[b], SEMI[b])
                pltpu.async_copy(dst_hbm.at[pl.ds(off2, B)], DIDXG[b], SEMI[b])
            _compute(b)
            sc1 = pltpu.async_copy(XLB[b], acc_sh.at[DIDXS[b]], SEMS[b],
                                   add=True)
            sc2 = pltpu.async_copy(DRB[b], den_sh.at[DIDXS[b]], SEMS[b],
                                   add=True)
            sc1.wait()
            sc2.wait()
            @pl.when(kk < khalf - 1)
            def _():
                pltpu.make_async_copy(src_hbm.at[pl.ds(off2, B)], SIDX[b],
                                      SEMI[b]).wait()
                pltpu.make_async_copy(dst_hbm.at[pl.ds(off2, B)], DIDXG[b],
                                      SEMI[b]).wait()
                _issue_gather(b, None)
        return 0
    lax.fori_loop(0, khalf, _kk, 0)

    plsc.subcore_barrier()
    for t in range(10):
        sz = 64 if t < 9 else 56
        r0 = base_row + t * 64
        pltpu.sync_copy(acc_sh.at[pl.ds(r0, sz)], xlb0.at[pl.ds(0, sz)])
        pltpu.sync_copy(xlb0.at[pl.ds(0, sz)], acc_out.at[cid, pl.ds(r0, sz)])
        pltpu.sync_copy(den_sh.at[pl.ds(r0, sz)], drb0.at[pl.ds(0, sz)])
        pltpu.sync_copy(drb0.at[pl.ds(0, sz)], den_out.at[cid, pl.ds(r0, sz)])


def _sc_aggregate(xl, xr, src, dst, att_flat):
    mesh = plsc.VectorSubcoreMesh(core_axis_name="c", subcore_axis_name="s")
    return pl.kernel(
        _sc_body,
        out_type=[
            jax.ShapeDtypeStruct((NC, PAD_N, C), jnp.float32),
            jax.ShapeDtypeStruct((NC, PAD_N, L), jnp.float32),
        ],
        mesh=mesh,
        compiler_params=pltpu.CompilerParams(use_tc_tiling_on_sc=False),
        scratch_types=[
            pltpu.VMEM_SHARED((PAD_N, C), jnp.float32),
            pltpu.VMEM_SHARED((PAD_N, L), jnp.float32),
            pltpu.VMEM((B, C), jnp.float32),
            pltpu.VMEM((B, C), jnp.float32),
            pltpu.VMEM((B, C), jnp.float32),
            pltpu.VMEM((B, C), jnp.float32),
            pltpu.VMEM((B,), jnp.int32),
            pltpu.VMEM((B,), jnp.int32),
            pltpu.VMEM((B,), jnp.int32),
            pltpu.VMEM((B,), jnp.int32),
            pltpu.VMEM((B,), jnp.int32),
            pltpu.VMEM((B,), jnp.int32),
            pltpu.VMEM((B, L), jnp.float32),
            pltpu.VMEM((B, L), jnp.float32),
            pltpu.VMEM((C,), jnp.float32),
            pltpu.SemaphoreType.DMA,
            pltpu.SemaphoreType.DMA,
            pltpu.SemaphoreType.DMA,
            pltpu.SemaphoreType.DMA,
            pltpu.SemaphoreType.DMA,
            pltpu.SemaphoreType.DMA,
        ],
    )(xl, xr, src, dst, att_flat)


def _epilogue_body(acc_ref, den_ref, out_ref):
    num = acc_ref[0] + acc_ref[1]
    den = den_ref[0, :, 0:1] + den_ref[1, :, 0:1]
    out_ref[...] = num / (den + 1e-16)


def _epilogue(acc, den):
    blk = 1000
    return pl.pallas_call(
        _epilogue_body,
        grid=(N // blk,),
        in_specs=[
            pl.BlockSpec((NC, blk, C), lambda i: (0, i, 0)),
            pl.BlockSpec((NC, blk, L), lambda i: (0, i, 0)),
        ],
        out_specs=pl.BlockSpec((blk, C), lambda i: (i, 0)),
        out_shape=jax.ShapeDtypeStruct((N, C), jnp.float32),
    )(acc, den)


def kernel(x, edge_index, W_l, W_r, att):
    xl, xr = _project(x, W_l, W_r)
    npad = EPAD - (E + N)
    loop = jnp.arange(N, dtype=jnp.int32)
    src = jnp.concatenate([edge_index[0], loop,
                           jnp.zeros((npad,), jnp.int32)])
    dst = jnp.concatenate([edge_index[1], loop,
                           jnp.full((npad,), N, jnp.int32)])
    acc, den = _sc_aggregate(xl, xr, src, dst, att.reshape(C))
    return _epilogue(acc, den)


# speedup vs baseline: 1.0515x; 1.0515x over previous
import functools

import jax
import jax.numpy as jnp
from jax import lax
from jax.experimental import pallas as pl
from jax.experimental.pallas import tpu as pltpu, tpu_sc as plsc

N = 10000
E = 320000
C = 128

NC, NS, L = 2, 16, 16
NW = NC * NS
B = 64
K = 164
EPAD = NW * B * K
PAD_N = 10112
ROWS_PER_TILE = PAD_N // NS
_ZCHUNKS = (128, 128, 128, 128, 120)


def _project_body(x_ref, wl_ref, wr_ref, xl_ref, xr_ref):
    xb = x_ref[...]
    xl_ref[...] = jnp.dot(xb, wl_ref[...], preferred_element_type=jnp.float32)
    xr_ref[...] = jnp.dot(xb, wr_ref[...], preferred_element_type=jnp.float32)


def _project(x, W_l, W_r):
    blk = 1000
    return pl.pallas_call(
        _project_body,
        grid=(N // blk,),
        in_specs=[
            pl.BlockSpec((blk, C), lambda i: (i, 0)),
            pl.BlockSpec((C, C), lambda i: (0, 0)),
            pl.BlockSpec((C, C), lambda i: (0, 0)),
        ],
        out_specs=[
            pl.BlockSpec((blk, C), lambda i: (i, 0)),
            pl.BlockSpec((blk, C), lambda i: (i, 0)),
        ],
        out_shape=[
            jax.ShapeDtypeStruct((N, C), jnp.float32),
            jax.ShapeDtypeStruct((N, C), jnp.float32),
        ],
    )(x, W_l, W_r)


def _sc_body(xl_hbm, xr_hbm, src_hbm, dst_hbm, att_hbm,
             acc_out, den_out,
             acc_sh, den_sh,
             xlb0, xlb1, xrb0, xrb1,
             sidx0, sidx1, didxg0, didxg1, didxs0, didxs1,
             drb0, drb1, attv,
             sem_g0, sem_g1, sem_i0, sem_i1, sem_s0, sem_s1):
    cid = lax.axis_index("c")
    sid = lax.axis_index("s")
    wid = cid * NS + sid

    zv = jnp.zeros((L,), jnp.float32)
    lane = lax.iota(jnp.int32, L)

    XLB, XRB = (xlb0, xlb1), (xrb0, xrb1)
    SIDX, DIDXG, DIDXS = (sidx0, sidx1), (didxg0, didxg1), (didxs0, didxs1)
    DRB = (drb0, drb1)
    SEMG, SEMI, SEMS = (sem_g0, sem_g1), (sem_i0, sem_i1), (sem_s0, sem_s1)

    def _zero(r, _):
        for c in range(C // L):
            xlb0[r, pl.ds(c * L, L)] = zv
        drb0[r, :] = zv
        return 0
    lax.fori_loop(0, B, _zero, 0)
    base_row = sid * ROWS_PER_TILE
    for t in range(10):
        sz = 64 if t < 9 else 56
        pltpu.sync_copy(xlb0.at[pl.ds(0, sz)],
                        acc_sh.at[pl.ds(base_row + t * 64, sz)])
        pltpu.sync_copy(drb0.at[pl.ds(0, sz)],
                        den_sh.at[pl.ds(base_row + t * 64, sz)])
    pltpu.sync_copy(att_hbm, attv)
    plsc.subcore_barrier()

    att_regs = [attv[pl.ds(c * L, L)] for c in range(C // L)]
    ebase = wid * (K * B)

    def _issue_gather(b, off):
        pltpu.async_copy(xl_hbm.at[SIDX[b]], XLB[b], SEMG[b])
        pltpu.async_copy(xr_hbm.at[DIDXG[b]], XRB[b], SEMG[b])
        del off

    def _wait_gather(b):
        pltpu.make_async_copy(xl_hbm.at[SIDX[b]], XLB[b], SEMG[b]).wait()
        pltpu.make_async_copy(xr_hbm.at[DIDXG[b]], XRB[b], SEMG[b]).wait()

    for b in (0, 1):
        pltpu.sync_copy(src_hbm.at[pl.ds(ebase + b * B, B)], SIDX[b])
        pltpu.sync_copy(dst_hbm.at[pl.ds(ebase + b * B, B)], DIDXG[b])
        _issue_gather(b, None)

    def _compute(b):
        xlb, xrb, drb = XLB[b], XRB[b], DRB[b]

        def _edge(e, _):
            xl_r = [xlb[e, pl.ds(c * L, L)] for c in range(C // L)]
            acc = zv
            for c in range(C // L):
                v = xl_r[c] + xrb[e, pl.ds(c * L, L)]
                v = jnp.maximum(v, 0.2 * v)
                acc = acc + v * att_regs[c]
            for h in (8, 4, 2, 1):
                acc = acc + acc.at[lane ^ h].get(mode="promise_in_bounds")
            wv = jnp.exp(acc)
            for c in range(C // L):
                xlb[e, pl.ds(c * L, L)] = xl_r[c] * wv
            drb[e, :] = jnp.where(lane == 0, wv, 0.0)
            return 0
        lax.fori_loop(0, B, _edge, 0, unroll=4)

    def _kk(kk, _):
        for b in (0, 1):
            k = 2 * kk + b
            _wait_gather(b)
            for j in range(B // L):
                DIDXS[b][pl.ds(j * L, L)] = DIDXG[b][pl.ds(j * L, L)]
            off2 = ebase + (k + 2) * B
            @pl.when(kk < K // 2 - 1)
            def _():
                pltpu.async_copy(src_hbm.at[pl.ds(off2, B)], SIDX[b], SEMI[b])
                pltpu.async_copy(dst_hbm.at[pl.ds(off2, B)], DIDXG[b], SEMI[b])
            _compute(b)
            sc1 = pltpu.async_copy(XLB[b], acc_sh.at[DIDXS[b]], SEMS[b],
                                   add=True)
            sc2 = pltpu.async_copy(DRB[b], den_sh.at[DIDXS[b]], SEMS[b],
                                   add=True)
            sc1.wait()
            sc2.wait()
            @pl.when(kk < K // 2 - 1)
            def _():
                pltpu.make_async_copy(src_hbm.at[pl.ds(off2, B)], SIDX[b],
                                      SEMI[b]).wait()
                pltpu.make_async_copy(dst_hbm.at[pl.ds(off2, B)], DIDXG[b],
                                      SEMI[b]).wait()
                _issue_gather(b, None)
        return 0
    lax.fori_loop(0, K // 2, _kk, 0)

    plsc.subcore_barrier()
    for t in range(10):
        sz = 64 if t < 9 else 56
        r0 = base_row + t * 64
        pltpu.sync_copy(acc_sh.at[pl.ds(r0, sz)], xlb0.at[pl.ds(0, sz)])
        pltpu.sync_copy(xlb0.at[pl.ds(0, sz)], acc_out.at[cid, pl.ds(r0, sz)])
        pltpu.sync_copy(den_sh.at[pl.ds(r0, sz)], drb0.at[pl.ds(0, sz)])
        pltpu.sync_copy(drb0.at[pl.ds(0, sz)], den_out.at[cid, pl.ds(r0, sz)])


def _sc_aggregate(xl, xr, src, dst, att_flat):
    mesh = plsc.VectorSubcoreMesh(core_axis_name="c", subcore_axis_name="s")
    return pl.kernel(
        _sc_body,
        out_type=[
            jax.ShapeDtypeStruct((NC, PAD_N, C), jnp.float32),
            jax.ShapeDtypeStruct((NC, PAD_N, L), jnp.float32),
        ],
        mesh=mesh,
        compiler_params=pltpu.CompilerParams(use_tc_tiling_on_sc=False),
        scratch_types=[
            pltpu.VMEM_SHARED((PAD_N, C), jnp.float32),
            pltpu.VMEM_SHARED((PAD_N, L), jnp.float32),
            pltpu.VMEM((B, C), jnp.float32),
            pltpu.VMEM((B, C), jnp.float32),
            pltpu.VMEM((B, C), jnp.float32),
            pltpu.VMEM((B, C), jnp.float32),
            pltpu.VMEM((B,), jnp.int32),
            pltpu.VMEM((B,), jnp.int32),
            pltpu.VMEM((B,), jnp.int32),
            pltpu.VMEM((B,), jnp.int32),
            pltpu.VMEM((B,), jnp.int32),
            pltpu.VMEM((B,), jnp.int32),
            pltpu.VMEM((B, L), jnp.float32),
            pltpu.VMEM((B, L), jnp.float32),
            pltpu.VMEM((C,), jnp.float32),
            pltpu.SemaphoreType.DMA,
            pltpu.SemaphoreType.DMA,
            pltpu.SemaphoreType.DMA,
            pltpu.SemaphoreType.DMA,
            pltpu.SemaphoreType.DMA,
            pltpu.SemaphoreType.DMA,
        ],
    )(xl, xr, src, dst, att_flat)


def _epilogue_body(acc_ref, den_ref, out_ref):
    num = acc_ref[0] + acc_ref[1]
    den = den_ref[0, :, 0:1] + den_ref[1, :, 0:1]
    out_ref[...] = num / (den + 1e-16)


def _epilogue(acc, den):
    blk = 1000
    return pl.pallas_call(
        _epilogue_body,
        grid=(N // blk,),
        in_specs=[
            pl.BlockSpec((NC, blk, C), lambda i: (0, i, 0)),
            pl.BlockSpec((NC, blk, L), lambda i: (0, i, 0)),
        ],
        out_specs=pl.BlockSpec((blk, C), lambda i: (i, 0)),
        out_shape=jax.ShapeDtypeStruct((N, C), jnp.float32),
    )(acc, den)


def kernel(x, edge_index, W_l, W_r, att):
    xl, xr = _project(x, W_l, W_r)
    npad = EPAD - (E + N)
    loop = jnp.arange(N, dtype=jnp.int32)
    src = jnp.concatenate([edge_index[0], loop,
                           jnp.zeros((npad,), jnp.int32)])
    dst = jnp.concatenate([edge_index[1], loop,
                           jnp.full((npad,), N, jnp.int32)])
    acc, den = _sc_aggregate(xl, xr, src, dst, att.reshape(C))
    return _epilogue(acc, den)
